# baseline (speedup 1.0000x reference)
import jax
import jax.numpy as jnp
from jax.experimental import pallas as pl

_PROMPT_ROWS = 101
_BATCH_BLOCK = 128


def _body(emb_ref, out_ref, mask_ref):
    out_ref[...] = jnp.broadcast_to(
        emb_ref[...][None, :, :], out_ref.shape)
    mask_ref[...] = jnp.ones(mask_ref.shape, jnp.float32)


def kernel(feature_map, key, embedding):
    del key
    batch = feature_map.shape[1]
    embed_dim = embedding.shape[1]
    emb = embedding[:_PROMPT_ROWS]

    tiled, mask = pl.pallas_call(
        _body,
        grid=(batch // _BATCH_BLOCK,),
        in_specs=[pl.BlockSpec((_PROMPT_ROWS, embed_dim), lambda i: (0, 0))],
        out_specs=[
            pl.BlockSpec((_BATCH_BLOCK, _PROMPT_ROWS, embed_dim),
                         lambda i: (i, 0, 0)),
            pl.BlockSpec((_BATCH_BLOCK, _PROMPT_ROWS), lambda i: (i, 0)),
        ],
        out_shape=[
            jax.ShapeDtypeStruct((batch, _PROMPT_ROWS, embed_dim), jnp.float32),
            jax.ShapeDtypeStruct((batch, _PROMPT_ROWS), jnp.float32),
        ],
    )(emb)
    return (tiled, mask)


# speedup vs baseline: 1.0123x; 1.0123x over previous
import jax
import jax.numpy as jnp
from jax.experimental import pallas as pl
from jax.experimental.pallas import tpu as pltpu

_PROMPT_ROWS = 101
_CHUNK = 128
_NSEM = 8


def _body(emb_ref, out_ref, mask_ref, ebuf, mbuf, esem, msem):
    ebuf[...] = jnp.broadcast_to(emb_ref[...][None, :, :], ebuf.shape)
    mbuf[...] = jnp.ones(mbuf.shape, jnp.float32)
    nchunks = out_ref.shape[0] // _CHUNK
    copies = []
    for i in range(nchunks):
        if i >= _NSEM:
            copies[i - _NSEM][0].wait()
            copies[i - _NSEM][1].wait()
        c = pltpu.make_async_copy(
            ebuf, out_ref.at[pl.ds(i * _CHUNK, _CHUNK)], esem.at[i % _NSEM])
        m = pltpu.make_async_copy(
            mbuf, mask_ref.at[pl.ds(i * _CHUNK, _CHUNK)], msem.at[i % _NSEM])
        c.start()
        m.start()
        copies.append((c, m))
    for i in range(max(0, nchunks - _NSEM), nchunks):
        copies[i][0].wait()
        copies[i][1].wait()


def kernel(feature_map, key, embedding):
    del key
    batch = feature_map.shape[1]
    embed_dim = embedding.shape[1]
    emb = embedding[:_PROMPT_ROWS]

    tiled, mask = pl.pallas_call(
        _body,
        in_specs=[pl.BlockSpec(memory_space=pltpu.MemorySpace.VMEM)],
        out_specs=[
            pl.BlockSpec(memory_space=pl.ANY),
            pl.BlockSpec(memory_space=pl.ANY),
        ],
        out_shape=[
            jax.ShapeDtypeStruct((batch, _PROMPT_ROWS, embed_dim), jnp.float32),
            jax.ShapeDtypeStruct((batch, _PROMPT_ROWS), jnp.float32),
        ],
        scratch_shapes=[
            pltpu.VMEM((_CHUNK, _PROMPT_ROWS, embed_dim), jnp.float32),
            pltpu.VMEM((_CHUNK, _PROMPT_ROWS), jnp.float32),
            pltpu.SemaphoreType.DMA((_NSEM,)),
            pltpu.SemaphoreType.DMA((_NSEM,)),
        ],
    )(emb)
    return (tiled, mask)
